# Initial kernel scaffold; baseline (speedup 1.0000x reference)
import jax
import jax.numpy as jnp
from jax.experimental import pallas as pl


def kernel(x, edge_index, batch, W1, b1, W2, b2):
    raise NotImplementedError("write your pallas kernel here")


# speedup vs baseline: 19.3168x; 19.3168x over previous
import functools

import jax
import jax.numpy as jnp
from jax import lax
from jax.experimental import pallas as pl
from jax.experimental.pallas import tpu as pltpu
from jax.experimental.pallas import tpu_sc as plsc

_NC = 2
_NS = 16
_NW = _NC * _NS
_G = 128
_K = 80


def _sc_mesh():
  return plsc.VectorSubcoreMesh(core_axis_name="c", subcore_axis_name="s",
                                num_cores=_NC, num_subcores=_NS)


@functools.cache
def _deg_kernel(E, NP):
  epw = E // _NW
  nchunks = epw // _K
  rpt = NP // _NS

  @functools.partial(
      pl.kernel,
      out_type=jax.ShapeDtypeStruct((_NC, NP), jnp.float32),
      mesh=_sc_mesh(),
      scratch_types=[
          pltpu.VMEM((nchunks, _K), jnp.int32),
          pltpu.VMEM((_K,), jnp.float32),
          pltpu.VMEM((rpt,), jnp.float32),
          pltpu.VMEM_SHARED((NP,), jnp.float32),
      ],
  )
  def deg(dst_hbm, out_hbm, idx_v, ones_v, zbuf, acc):
    cid = lax.axis_index("c")
    sid = lax.axis_index("s")
    wid = sid * _NC + cid
    r0 = sid * rpt

    @pl.loop(0, rpt // 16)
    def _z(i):
      zbuf[pl.ds(i * 16, 16)] = jnp.zeros((16,), jnp.float32)

    @pl.loop(0, _K // 16)
    def _o(i):
      ones_v[pl.ds(i * 16, 16)] = jnp.ones((16,), jnp.float32)

    pltpu.sync_copy(dst_hbm.at[wid], idx_v)
    pltpu.sync_copy(zbuf, acc.at[pl.ds(r0, rpt)])
    plsc.subcore_barrier()

    @pl.loop(0, nchunks)
    def _c(c):
      pltpu.sync_copy(ones_v, acc.at[idx_v.at[c]], add=True)

    plsc.subcore_barrier()
    pltpu.sync_copy(acc.at[pl.ds(r0, rpt)], out_hbm.at[cid, pl.ds(r0, rpt)])

  return deg


@functools.cache
def _agg_kernel(E, NP, F):
  epw = E // _NW
  nchunks = epw // _K
  rpt = NP // _NS

  @functools.partial(
      pl.kernel,
      out_type=jax.ShapeDtypeStruct((_NC, NP, F), jnp.float32),
      mesh=_sc_mesh(),
      scratch_types=[
          pltpu.VMEM((nchunks, _K), jnp.int32),
          pltpu.VMEM((nchunks, _K), jnp.int32),
          pltpu.VMEM((_K, F), jnp.float32),
          pltpu.VMEM_SHARED((NP, F), jnp.float32),
          pltpu.SemaphoreType.DMA,
      ],
  )
  def agg(rows_hbm, src_hbm, dst_hbm, out_hbm, src_v, dst_v, rows_v, acc, sem):
    cid = lax.axis_index("c")
    sid = lax.axis_index("s")
    wid = sid * _NC + cid
    r0 = sid * rpt

    pltpu.sync_copy(src_hbm.at[wid], src_v)
    pltpu.sync_copy(dst_hbm.at[wid], dst_v)
    pltpu.sync_copy(rows_hbm.at[pl.ds(r0, rpt)], acc.at[pl.ds(r0, rpt)])
    plsc.subcore_barrier()

    @pl.loop(0, nchunks)
    def _c(c):
      pltpu.async_copy(rows_hbm.at[src_v.at[c]], rows_v, sem).wait()
      pltpu.sync_copy(rows_v, acc.at[dst_v.at[c]], add=True)

    plsc.subcore_barrier()
    pltpu.sync_copy(acc.at[pl.ds(r0, rpt)], out_hbm.at[cid, pl.ds(r0, rpt)])

  return agg


def _tc_a(xp, W1, deg2, NP):
  D = xp.shape[1]
  H = W1.shape[1]
  BR = 640
  NB = NP // BR

  def body(x_r, w_r, d_r, h_r, dv_r):
    dv = lax.rsqrt(d_r[0] + d_r[1] + 1.0)
    h = jnp.dot(x_r[...], w_r[...], preferred_element_type=jnp.float32)
    h_r[...] = h * dv
    dv_r[...] = dv

  return pl.pallas_call(
      body,
      grid=(NB,),
      in_specs=[
          pl.BlockSpec((BR, D), lambda i: (i, 0)),
          pl.BlockSpec((D, H), lambda i: (0, 0)),
          pl.BlockSpec((2, BR, 1), lambda i: (0, i, 0)),
      ],
      out_specs=[
          pl.BlockSpec((BR, H), lambda i: (i, 0)),
          pl.BlockSpec((BR, 1), lambda i: (i, 0)),
      ],
      out_shape=[
          jax.ShapeDtypeStruct((NP, H), jnp.float32),
          jax.ShapeDtypeStruct((NP, 1), jnp.float32),
      ],
  )(xp, W1, deg2)


def _tc_b(p, h1s, dinv, b1r, W2p, NP):
  H = h1s.shape[1]
  F2 = W2p.shape[1]
  BR = 640
  NB = NP // BR

  def body(p_r, h_r, dv_r, b1_r, w2_r, o_r):
    dv = dv_r[...]
    z = jnp.maximum(dv * (p_r[0] + p_r[1] - h_r[...]) + b1_r[...], 0.0)
    o_r[...] = jnp.dot(z, w2_r[...], preferred_element_type=jnp.float32) * dv

  return pl.pallas_call(
      body,
      grid=(NB,),
      in_specs=[
          pl.BlockSpec((2, BR, H), lambda i: (0, i, 0)),
          pl.BlockSpec((BR, H), lambda i: (i, 0)),
          pl.BlockSpec((BR, 1), lambda i: (i, 0)),
          pl.BlockSpec((1, H), lambda i: (0, 0)),
          pl.BlockSpec((H, F2), lambda i: (0, 0)),
      ],
      out_specs=pl.BlockSpec((BR, F2), lambda i: (i, 0)),
      out_shape=jax.ShapeDtypeStruct((NP, F2), jnp.float32),
  )(p, h1s, dinv, b1r, W2p)


def _tc_c(q, h2s, dinv, b2p, pmask, batch3, N):
  F2 = h2s.shape[1]
  BR = 1000
  NB = N // BR

  def body(q_r, h_r, dv_r, b2_r, pm_r, bt_r, o_r, sums, cnts):
    i = pl.program_id(0)

    @pl.when(i == 0)
    def _():
      sums[...] = jnp.zeros_like(sums)
      cnts[...] = jnp.zeros_like(cnts)

    out2 = dv_r[...] * (q_r[0] + q_r[1] - h_r[...]) + b2_r[...]
    bi = bt_r[0, 0, :]
    oh = (bi[:, None] == lax.broadcasted_iota(jnp.int32, (BR, _G), 1)
          ).astype(jnp.float32)
    sums[...] += lax.dot_general(oh, out2, (((0,), (0,)), ((), ())),
                                 preferred_element_type=jnp.float32)
    cnts[...] += lax.dot_general(oh, jnp.ones((BR, 1), jnp.float32),
                                 (((0,), (0,)), ((), ())),
                                 preferred_element_type=jnp.float32)

    @pl.when(i == NB - 1)
    def _():
      m = sums[...] / jnp.maximum(cnts[...], 1.0) + pm_r[...]
      mx = jnp.max(m, axis=1, keepdims=True)
      lse = jnp.log(jnp.sum(jnp.exp(m - mx), axis=1, keepdims=True)) + mx
      o_r[...] = m - lse

  return pl.pallas_call(
      body,
      grid=(NB,),
      in_specs=[
          pl.BlockSpec((2, BR, F2), lambda i: (0, i, 0)),
          pl.BlockSpec((BR, F2), lambda i: (i, 0)),
          pl.BlockSpec((BR, 1), lambda i: (i, 0)),
          pl.BlockSpec((1, F2), lambda i: (0, 0)),
          pl.BlockSpec((1, F2), lambda i: (0, 0)),
          pl.BlockSpec((1, 1, BR), lambda i: (i, 0, 0)),
      ],
      out_specs=pl.BlockSpec((_G, F2), lambda i: (0, 0)),
      out_shape=jax.ShapeDtypeStruct((_G, F2), jnp.float32),
      scratch_shapes=[
          pltpu.VMEM((_G, F2), jnp.float32),
          pltpu.VMEM((_G, 1), jnp.float32),
      ],
  )(q, h2s, dinv, b2p, pmask, batch3)


def kernel(x, edge_index, batch, W1, b1, W2, b2):
  N, D = x.shape
  H = W1.shape[1]
  C = W2.shape[1]
  E = edge_index.shape[1]
  F2 = 128
  NP = -(-N // 640) * 640

  src2 = edge_index[0].reshape(_NW, E // (_NW * _K), _K)
  dst2 = edge_index[1].reshape(_NW, E // (_NW * _K), _K)

  xp = jnp.pad(x, ((0, NP - N), (0, 0)))
  deg = _deg_kernel(E, NP)(dst2)
  deg2 = deg.reshape(2, NP, 1)
  h1s, dinv = _tc_a(xp, W1, deg2, NP)

  p = _agg_kernel(E, NP, H)(h1s, src2, dst2)
  b1r = b1.reshape(1, H)
  W2p = jnp.pad(W2, ((0, 0), (0, F2 - C)))
  h2s = _tc_b(p, h1s, dinv, b1r, W2p, NP)

  q = _agg_kernel(E, NP, F2)(h2s, src2, dst2)
  b2p = jnp.pad(b2, (0, F2 - C)).reshape(1, F2)
  pmask = jnp.concatenate(
      [jnp.zeros((C,), jnp.float32),
       jnp.full((F2 - C,), -1e9, jnp.float32)]).reshape(1, F2)
  batch3 = batch.reshape(N // 1000, 1, 1000)
  out = _tc_c(q, h2s, dinv, b2p, pmask, batch3, N)
  return out[:, :C]
